# baseline (speedup 1.0000x reference)
import jax
import jax.numpy as jnp
from jax.experimental import pallas as pl

BATCH = 4
SEQ_LEN = 2048
EMBED_DIM = 768


def _add_kernel(x_ref, pos_ref, o_ref):
    o_ref[...] = x_ref[...] + pos_ref[...]


def kernel(x, pos_table):
    grid = (2,)
    return pl.pallas_call(
        _add_kernel,
        grid=grid,
        in_specs=[
            pl.BlockSpec((BATCH, SEQ_LEN // 2, EMBED_DIM), lambda s: (0, s, 0)),
            pl.BlockSpec((SEQ_LEN // 2, EMBED_DIM), lambda s: (s, 0)),
        ],
        out_specs=pl.BlockSpec((BATCH, SEQ_LEN // 2, EMBED_DIM), lambda s: (0, s, 0)),
        out_shape=jax.ShapeDtypeStruct(x.shape, x.dtype),
    )(x, pos_table)


# speedup vs baseline: 1.0669x; 1.0669x over previous
import jax
import jax.numpy as jnp
from jax.experimental import pallas as pl

BATCH = 4
SEQ_LEN = 2048
EMBED_DIM = 768


def _add_kernel(x_ref, pos_ref, o_ref):
    o_ref[...] = x_ref[...] + pos_ref[...]


def kernel(x, pos_table):
    grid = (BATCH // 2,)
    return pl.pallas_call(
        _add_kernel,
        grid=grid,
        in_specs=[
            pl.BlockSpec((2, SEQ_LEN, EMBED_DIM), lambda b: (b, 0, 0)),
            pl.BlockSpec((SEQ_LEN, EMBED_DIM), lambda b: (0, 0)),
        ],
        out_specs=pl.BlockSpec((2, SEQ_LEN, EMBED_DIM), lambda b: (b, 0, 0)),
        out_shape=jax.ShapeDtypeStruct(x.shape, x.dtype),
    )(x, pos_table)
